# Initial kernel scaffold; baseline (speedup 1.0000x reference)
import jax
import jax.numpy as jnp
from jax.experimental import pallas as pl


def kernel(x, edge_index, W1, a_src1, a_dst1, b1, W2, a_src2, a_dst2, b2):
    raise NotImplementedError("write your pallas kernel here")


# speedup vs baseline: 32.2322x; 32.2322x over previous
import functools

import jax
import jax.numpy as jnp
from jax import lax
from jax.experimental import pallas as pl
from jax.experimental.pallas import tpu as pltpu
from jax.experimental.pallas import tpu_sc as plsc

N = 10000
E = 320000
DIM_IN = 128
HID = 16
HEADS = 8
NUM_CLASSES = 40
NEG_SLOPE = 0.2
NCP = 48
NG = 3

NC = 2
NS = 16
L = 16


BLK = 2000


def _tc_a1_body(x_ref, w1_ref, as_ref, ad_ref, h1_ref, asrc_ref, adst_ref):
    h = jnp.dot(x_ref[...], w1_ref[...], preferred_element_type=jnp.float32)
    h1_ref[...] = h
    asrc_ref[...] = jnp.dot(h, as_ref[...], preferred_element_type=jnp.float32)
    adst_ref[...] = jnp.dot(h, ad_ref[...], preferred_element_type=jnp.float32)


def _tc_a1(x, w1, a_s, a_d):
    return pl.pallas_call(
        _tc_a1_body,
        grid=(N // BLK,),
        in_specs=[
            pl.BlockSpec((BLK, DIM_IN), lambda i: (i, 0)),
            pl.BlockSpec((DIM_IN, DIM_IN), lambda i: (0, 0)),
            pl.BlockSpec((DIM_IN, HEADS), lambda i: (0, 0)),
            pl.BlockSpec((DIM_IN, HEADS), lambda i: (0, 0)),
        ],
        out_specs=[
            pl.BlockSpec((BLK, HEADS * HID), lambda i: (i, 0)),
            pl.BlockSpec((BLK, HEADS), lambda i: (i, 0)),
            pl.BlockSpec((BLK, HEADS), lambda i: (i, 0)),
        ],
        out_shape=(
            jax.ShapeDtypeStruct((N, HEADS * HID), jnp.float32),
            jax.ShapeDtypeStruct((N, HEADS), jnp.float32),
            jax.ShapeDtypeStruct((N, HEADS), jnp.float32),
        ),
    )(x, w1, a_s, a_d)


def _tc_a2_body(asrc_ref, adst_ref, pack_ref):
    a_s = asrc_ref[...].T
    a_d = adst_ref[...].T
    s_max = jnp.max(a_s, axis=1, keepdims=True)
    pre = s_max + a_d
    m = jnp.where(pre > 0, pre, NEG_SLOPE * pre)
    pack_ref[:, 0, :] = a_s
    pack_ref[:, 1, :] = a_d
    pack_ref[:, 2, :] = m


def _tc_a2(asrcT, adstT):
    return pl.pallas_call(
        _tc_a2_body,
        out_shape=jax.ShapeDtypeStruct((HEADS, 3, N), jnp.float32),
    )(asrcT, adstT)


def _tc_b1_body(num1_ref, den1p_ref, b1_ref, w2p_ref, was2_ref, wad2_ref,
                h2_ref, as2_ref, ad2_ref):
    b1 = b1_ref[...]
    was2 = was2_ref[...]
    wad2 = wad2_ref[...]
    acc = jnp.zeros((BLK, NCP), jnp.float32)
    as2 = jnp.zeros((1, 1, BLK), jnp.float32)
    ad2 = jnp.zeros((1, 1, BLK), jnp.float32)
    for hi in range(HEADS):
        den_h = (den1p_ref[0, hi] + den1p_ref[0, hi + 8]
                 + den1p_ref[0, hi + 16] + den1p_ref[0, hi + 24])
        z = num1_ref[hi] / (den_h[:, None] + 1e-16) + b1[hi][None, :]
        z = jnp.where(z > 0, z, jnp.exp(jnp.minimum(z, 0.0)) - 1.0)
        acc = acc + jnp.dot(z, w2p_ref[hi * HID:(hi + 1) * HID, :],
                            preferred_element_type=jnp.float32)
        as2 = as2 + (z * was2[hi][None, :]).sum(axis=1)[None, None, :]
        ad2 = ad2 + (z * wad2[hi][None, :]).sum(axis=1)[None, None, :]
    h2_ref[...] = acc
    as2_ref[...] = as2
    ad2_ref[...] = ad2


def _tc_b1(num1, den1p, b1, w2p, was2, wad2):
    return pl.pallas_call(
        _tc_b1_body,
        grid=(N // BLK,),
        in_specs=[
            pl.BlockSpec((HEADS, BLK, HID), lambda i: (0, i, 0)),
            pl.BlockSpec((1, NC * NS, BLK), lambda i: (i, 0, 0)),
            pl.BlockSpec((HEADS, HID), lambda i: (0, 0)),
            pl.BlockSpec((HEADS * HID, NCP), lambda i: (0, 0)),
            pl.BlockSpec((HEADS, HID), lambda i: (0, 0)),
            pl.BlockSpec((HEADS, HID), lambda i: (0, 0)),
        ],
        out_specs=[
            pl.BlockSpec((BLK, NCP), lambda i: (i, 0)),
            pl.BlockSpec((1, 1, BLK), lambda i: (i, 0, 0)),
            pl.BlockSpec((1, 1, BLK), lambda i: (i, 0, 0)),
        ],
        out_shape=(
            jax.ShapeDtypeStruct((N, NCP), jnp.float32),
            jax.ShapeDtypeStruct((N // BLK, 1, BLK), jnp.float32),
            jax.ShapeDtypeStruct((N // BLK, 1, BLK), jnp.float32),
        ),
    )(num1, den1p, b1, w2p, was2, wad2)


def _tc_b2_body(as2_ref, ad2_ref, pack2_ref):
    a_s = as2_ref[...]
    a_d = ad2_ref[...]
    s_max = jnp.max(a_s)
    pre = s_max + a_d
    m2 = jnp.where(pre > 0, pre, NEG_SLOPE * pre)
    pack2_ref[0, :] = a_s[0]
    pack2_ref[1, :] = a_d[0]
    pack2_ref[2, :] = m2[0]


def _tc_b2(as2, ad2):
    return pl.pallas_call(
        _tc_b2_body,
        out_shape=jax.ShapeDtypeStruct((3, N), jnp.float32),
    )(as2, ad2)


def _tc_c_body(num2_ref, den2p_ref, b2_ref, out_ref):
    den2 = jnp.sum(den2p_ref[...], axis=0)
    num = num2_ref[0] + num2_ref[1]
    out = num[:, :NUM_CLASSES] / (den2[:, None] + 1e-16) + b2_ref[...]
    out_ref[...] = out


def _tc_c(num2, den2p, b2):
    return pl.pallas_call(
        _tc_c_body,
        out_shape=jax.ShapeDtypeStruct((N, NUM_CLASSES), jnp.float32),
    )(num2, den2p, b2)


CH1 = 2000
NCHUNK1 = 40
EPT1 = E // 4


def _sc1_body(h1v, pack1, esrc, edst, zrows, zn, num1, den1p,
              pack_b, den_b, src_b, dst_b, idxg_b, dsti_b, ex_b, rows_b,
              numsh, sem):
    c = lax.axis_index("c")
    s = lax.axis_index("s")
    w = s * NC + c
    head = w % HEADS
    hs = head // NC
    chunkid = w // HEADS

    pltpu.sync_copy(pack1.at[head], pack_b)
    pltpu.sync_copy(zn, den_b)
    pltpu.sync_copy(zrows, numsh.at[pl.ds(s * 2500, 2500)])
    plsc.subcore_barrier()

    iota = lax.iota(jnp.int32, L)
    base_e = chunkid * EPT1

    def chunk_body(ci, _):
        eb = base_e + ci * CH1
        pltpu.sync_copy(esrc.at[pl.ds(eb, CH1)], src_b)
        pltpu.sync_copy(edst.at[pl.ds(eb, CH1)], dst_b)

        def grp(i, _):
            o = i * L
            sv = src_b[pl.ds(o, L)]
            dv = dst_b[pl.ds(o, L)]
            a_s = plsc.load_gather(pack_b, [sv])
            a_d = plsc.load_gather(pack_b, [dv + N])
            m_v = plsc.load_gather(pack_b, [dv + 2 * N])
            e = a_s + a_d
            e = jnp.where(e > 0, e, NEG_SLOPE * e)
            ex = jnp.exp(e - m_v)
            ex_b[pl.ds(o, L)] = ex
            idxg_b[pl.ds(o, L)] = sv * HEADS + head
            dsti_b[pl.ds(o, L)] = dv + hs * N
            plsc.addupdate_scatter(den_b, [dv], ex)
            return 0

        lax.fori_loop(0, CH1 // L, grp, 0)
        pltpu.async_copy(h1v.at[idxg_b], rows_b, sem).wait()

        def scale(i, _):
            pv = iota + i * L
            ex = ex_b[pl.ds(i * L, L)]
            for j in range(HID):
                jv = jnp.full((L,), j, jnp.int32)
                vals = plsc.load_gather(rows_b, [pv, jv])
                plsc.store_scatter(rows_b, [pv, jv], vals * ex)
            return 0

        lax.fori_loop(0, CH1 // L, scale, 0)
        pltpu.sync_copy(rows_b, numsh.at[dsti_b], add=True)
        return 0

    lax.fori_loop(0, NCHUNK1, chunk_body, 0)
    pltpu.sync_copy(den_b, den1p.at[w])
    plsc.subcore_barrier()

    @pl.when(w < HEADS)
    def _():
        pltpu.sync_copy(numsh.at[pl.ds(hs * N, N)], num1.at[head])


def _sc1(h1v, pack1, esrc, edst, zrows, zn):
    mesh = plsc.VectorSubcoreMesh(core_axis_name="c", subcore_axis_name="s",
                                  num_cores=NC, num_subcores=NS)
    return pl.kernel(
        _sc1_body,
        out_type=(
            jax.ShapeDtypeStruct((HEADS, N, HID), jnp.float32),
            jax.ShapeDtypeStruct((NC * NS, N), jnp.float32),
        ),
        mesh=mesh,
        scratch_types=[
            pltpu.VMEM((3 * N,), jnp.float32),
            pltpu.VMEM((N,), jnp.float32),
            pltpu.VMEM((CH1,), jnp.int32),
            pltpu.VMEM((CH1,), jnp.int32),
            pltpu.VMEM((CH1,), jnp.int32),
            pltpu.VMEM((CH1,), jnp.int32),
            pltpu.VMEM((CH1,), jnp.float32),
            pltpu.VMEM((CH1, HID), jnp.float32),
            pltpu.VMEM_SHARED((4 * N, HID), jnp.float32),
            pltpu.SemaphoreType.DMA,
        ],
        compiler_params=pltpu.CompilerParams(needs_layout_passes=False, use_tc_tiling_on_sc=False),
    )(h1v, pack1, esrc, edst, zrows, zn)


CH2 = 400
NCHUNK2 = 25
EPT2 = E // (NC * NS)


def _sc2_body(h2v, pack2, esrc, edst, zrows, zn, num2, den2p,
              pack_b, den_b, src_b, dst_b, idxg_b, dsti_b, ex_b, rows_b,
              numsh, sem):
    c = lax.axis_index("c")
    s = lax.axis_index("s")
    w = s * NC + c

    pltpu.sync_copy(pack2, pack_b)
    pltpu.sync_copy(zn, den_b)
    pltpu.sync_copy(zrows.at[pl.ds(0, 1875)], numsh.at[pl.ds(s * 1875, 1875)])
    plsc.subcore_barrier()

    iota = lax.iota(jnp.int32, L)
    base_e = w * EPT2

    def chunk_body(ci, _):
        eb = base_e + ci * CH2
        pltpu.sync_copy(esrc.at[pl.ds(eb, CH2)], src_b)
        pltpu.sync_copy(edst.at[pl.ds(eb, CH2)], dst_b)

        def grp(i, _):
            o = i * L
            sv = src_b[pl.ds(o, L)]
            dv = dst_b[pl.ds(o, L)]
            a_s = plsc.load_gather(pack_b, [sv])
            a_d = plsc.load_gather(pack_b, [dv + N])
            m_v = plsc.load_gather(pack_b, [dv + 2 * N])
            e = a_s + a_d
            e = jnp.where(e > 0, e, NEG_SLOPE * e)
            ex = jnp.exp(e - m_v)
            ex_b[pl.ds(o, L)] = ex
            for g in range(NG):
                idxg_b[pl.ds(g * CH2 + o, L)] = sv * NG + g
                dsti_b[pl.ds(g * CH2 + o, L)] = dv * NG + g
            plsc.addupdate_scatter(den_b, [dv], ex)
            return 0

        lax.fori_loop(0, CH2 // L, grp, 0)
        pltpu.async_copy(h2v.at[idxg_b], rows_b, sem).wait()

        def scale(i, _):
            ex = ex_b[pl.ds(i * L, L)]
            for g in range(NG):
                pv = iota + (g * CH2 + i * L)
                for j in range(HID):
                    jv = jnp.full((L,), j, jnp.int32)
                    vals = plsc.load_gather(rows_b, [pv, jv])
                    plsc.store_scatter(rows_b, [pv, jv], vals * ex)
            return 0

        lax.fori_loop(0, CH2 // L, scale, 0)
        pltpu.sync_copy(rows_b, numsh.at[dsti_b], add=True)
        return 0

    lax.fori_loop(0, NCHUNK2, chunk_body, 0)
    pltpu.sync_copy(den_b, den2p.at[w])
    plsc.subcore_barrier()

    @pl.when(s == 0)
    def _():
        pltpu.sync_copy(numsh, num2.at[c])


def _sc2(h2v, pack2, esrc, edst, zrows, zn):
    mesh = plsc.VectorSubcoreMesh(core_axis_name="c", subcore_axis_name="s",
                                  num_cores=NC, num_subcores=NS)
    return pl.kernel(
        _sc2_body,
        out_type=(
            jax.ShapeDtypeStruct((NC, NG * N, HID), jnp.float32),
            jax.ShapeDtypeStruct((NC * NS, N), jnp.float32),
        ),
        mesh=mesh,
        scratch_types=[
            pltpu.VMEM((3 * N,), jnp.float32),
            pltpu.VMEM((N,), jnp.float32),
            pltpu.VMEM((CH2,), jnp.int32),
            pltpu.VMEM((CH2,), jnp.int32),
            pltpu.VMEM((NG * CH2,), jnp.int32),
            pltpu.VMEM((NG * CH2,), jnp.int32),
            pltpu.VMEM((CH2,), jnp.float32),
            pltpu.VMEM((NG * CH2, HID), jnp.float32),
            pltpu.VMEM_SHARED((NG * N, HID), jnp.float32),
            pltpu.SemaphoreType.DMA,
        ],
        compiler_params=pltpu.CompilerParams(needs_layout_passes=False, use_tc_tiling_on_sc=False),
    )(h2v, pack2, esrc, edst, zrows, zn)


def kernel(x, edge_index, W1, a_src1, a_dst1, b1, W2, a_src2, a_dst2, b2):
    esrc = edge_index[0]
    edst = edge_index[1]
    eye = jnp.eye(HEADS, dtype=jnp.float32)
    a_s = (a_src1.reshape(HEADS, HID)[:, None, :]
           * eye[:, :, None]).transpose(0, 2, 1).reshape(HEADS * HID, HEADS)
    a_d = (a_dst1.reshape(HEADS, HID)[:, None, :]
           * eye[:, :, None]).transpose(0, 2, 1).reshape(HEADS * HID, HEADS)
    zrows = jnp.zeros((2500, HID), jnp.float32)
    zn = jnp.zeros((N,), jnp.float32)

    h1, asrcT, adstT = _tc_a1(x, W1, a_s, a_d)
    pack1 = _tc_a2(asrcT, adstT)
    h1v = h1.reshape(N * HEADS, HID)
    num1, den1p = _sc1(h1v, pack1.reshape(HEADS, 3 * N), esrc, edst, zrows, zn)

    w2p = jnp.pad(W2, ((0, 0), (0, NCP - NUM_CLASSES)))
    was2 = (W2 @ a_src2.reshape(NUM_CLASSES)).reshape(HEADS, HID)
    wad2 = (W2 @ a_dst2.reshape(NUM_CLASSES)).reshape(HEADS, HID)
    den1pt = den1p.reshape(NC * NS, N // BLK, BLK).transpose(1, 0, 2)
    h2, as2, ad2 = _tc_b1(num1, den1pt, b1.reshape(HEADS, HID), w2p, was2,
                          wad2)
    pack2 = _tc_b2(as2.reshape(1, N), ad2.reshape(1, N))

    h2v = h2.reshape(N * NG, HID)
    num2, den2p = _sc2(h2v, pack2.reshape(3 * N), esrc, edst, zrows, zn)

    return _tc_c(num2.reshape(NC, N, NCP), den2p, b2.reshape(1, NUM_CLASSES))


# speedup vs baseline: 69.7467x; 2.1639x over previous
import functools

import jax
import jax.numpy as jnp
from jax import lax
from jax.experimental import pallas as pl
from jax.experimental.pallas import tpu as pltpu
from jax.experimental.pallas import tpu_sc as plsc

N = 10000
E = 320000
DIM_IN = 128
HID = 16
HEADS = 8
NUM_CLASSES = 40
NEG_SLOPE = 0.2
NCP = 48
NG = 3

NC = 2
NS = 16
L = 16


BLK = 2000


def _tc_a1_body(x_ref, w1_ref, as_ref, ad_ref, h1_ref, asrc_ref, adst_ref):
    h = jnp.dot(x_ref[...], w1_ref[...], preferred_element_type=jnp.float32)
    h1_ref[...] = h
    asrc_ref[...] = jnp.dot(h, as_ref[...], preferred_element_type=jnp.float32)
    adst_ref[...] = jnp.dot(h, ad_ref[...], preferred_element_type=jnp.float32)


def _tc_a1(x, w1, a_s, a_d):
    return pl.pallas_call(
        _tc_a1_body,
        grid=(N // BLK,),
        in_specs=[
            pl.BlockSpec((BLK, DIM_IN), lambda i: (i, 0)),
            pl.BlockSpec((DIM_IN, DIM_IN), lambda i: (0, 0)),
            pl.BlockSpec((DIM_IN, HEADS), lambda i: (0, 0)),
            pl.BlockSpec((DIM_IN, HEADS), lambda i: (0, 0)),
        ],
        out_specs=[
            pl.BlockSpec((BLK, HEADS * HID), lambda i: (i, 0)),
            pl.BlockSpec((BLK, HEADS), lambda i: (i, 0)),
            pl.BlockSpec((BLK, HEADS), lambda i: (i, 0)),
        ],
        out_shape=(
            jax.ShapeDtypeStruct((N, HEADS * HID), jnp.float32),
            jax.ShapeDtypeStruct((N, HEADS), jnp.float32),
            jax.ShapeDtypeStruct((N, HEADS), jnp.float32),
        ),
    )(x, w1, a_s, a_d)


def _tc_a2_body(asrc_ref, adst_ref, pack_ref):
    a_s = asrc_ref[...].T
    a_d = adst_ref[...].T
    s_max = jnp.max(a_s, axis=1, keepdims=True)
    pre = s_max + a_d
    m = jnp.where(pre > 0, pre, NEG_SLOPE * pre)
    pack_ref[:, 0, :] = a_s
    pack_ref[:, 1, :] = a_d
    pack_ref[:, 2, :] = m


def _tc_a2(asrcT, adstT):
    return pl.pallas_call(
        _tc_a2_body,
        out_shape=jax.ShapeDtypeStruct((HEADS, 3, N), jnp.float32),
    )(asrcT, adstT)


def _tc_b1_body(num1_ref, den1p_ref, b1_ref, w2p_ref, was2_ref, wad2_ref,
                h2_ref, as2_ref, ad2_ref):
    b1 = b1_ref[...]
    was2 = was2_ref[...]
    wad2 = wad2_ref[...]
    acc = jnp.zeros((BLK, NCP), jnp.float32)
    as2 = jnp.zeros((1, 1, BLK), jnp.float32)
    ad2 = jnp.zeros((1, 1, BLK), jnp.float32)
    for hi in range(HEADS):
        den_h = (den1p_ref[0, hi] + den1p_ref[0, hi + 8]
                 + den1p_ref[0, hi + 16] + den1p_ref[0, hi + 24])
        z = num1_ref[hi] / (den_h[:, None] + 1e-16) + b1[hi][None, :]
        z = jnp.where(z > 0, z, jnp.exp(jnp.minimum(z, 0.0)) - 1.0)
        acc = acc + jnp.dot(z, w2p_ref[hi * HID:(hi + 1) * HID, :],
                            preferred_element_type=jnp.float32)
        as2 = as2 + (z * was2[hi][None, :]).sum(axis=1)[None, None, :]
        ad2 = ad2 + (z * wad2[hi][None, :]).sum(axis=1)[None, None, :]
    h2_ref[...] = acc
    as2_ref[...] = as2
    ad2_ref[...] = ad2


def _tc_b1(num1, den1p, b1, w2p, was2, wad2):
    return pl.pallas_call(
        _tc_b1_body,
        grid=(N // BLK,),
        in_specs=[
            pl.BlockSpec((HEADS, BLK, HID), lambda i: (0, i, 0)),
            pl.BlockSpec((1, NC * NS, BLK), lambda i: (i, 0, 0)),
            pl.BlockSpec((HEADS, HID), lambda i: (0, 0)),
            pl.BlockSpec((HEADS * HID, NCP), lambda i: (0, 0)),
            pl.BlockSpec((HEADS, HID), lambda i: (0, 0)),
            pl.BlockSpec((HEADS, HID), lambda i: (0, 0)),
        ],
        out_specs=[
            pl.BlockSpec((BLK, NCP), lambda i: (i, 0)),
            pl.BlockSpec((1, 1, BLK), lambda i: (i, 0, 0)),
            pl.BlockSpec((1, 1, BLK), lambda i: (i, 0, 0)),
        ],
        out_shape=(
            jax.ShapeDtypeStruct((N, NCP), jnp.float32),
            jax.ShapeDtypeStruct((N // BLK, 1, BLK), jnp.float32),
            jax.ShapeDtypeStruct((N // BLK, 1, BLK), jnp.float32),
        ),
    )(num1, den1p, b1, w2p, was2, wad2)


def _tc_b2_body(as2_ref, ad2_ref, pack2_ref):
    a_s = as2_ref[...]
    a_d = ad2_ref[...]
    s_max = jnp.max(a_s)
    pre = s_max + a_d
    m2 = jnp.where(pre > 0, pre, NEG_SLOPE * pre)
    pack2_ref[0, :] = a_s[0]
    pack2_ref[1, :] = a_d[0]
    pack2_ref[2, :] = m2[0]


def _tc_b2(as2, ad2):
    return pl.pallas_call(
        _tc_b2_body,
        out_shape=jax.ShapeDtypeStruct((3, N), jnp.float32),
    )(as2, ad2)


def _tc_c_body(num2_ref, den2p_ref, b2_ref, out_ref):
    den2 = jnp.sum(den2p_ref[...], axis=0)
    num = num2_ref[0] + num2_ref[1]
    out = num[:, :NUM_CLASSES] / (den2[:, None] + 1e-16) + b2_ref[...]
    out_ref[...] = out


def _tc_c(num2, den2p, b2):
    return pl.pallas_call(
        _tc_c_body,
        out_shape=jax.ShapeDtypeStruct((N, NUM_CLASSES), jnp.float32),
    )(num2, den2p, b2)


CH1 = 800
NPAIR1 = 50
EPT1 = E // 4


def _sc1_body(h1v, pack1, esrc, edst, zrows, zn, num1, den1p,
              pack_b, den_b, src_b, dst_b,
              ex0, ex1, ig0, ig1, id0, id1, rows0, rows1,
              numsh, sem0, sem1):
    c = lax.axis_index("c")
    s = lax.axis_index("s")
    w = s * NC + c
    head = w % HEADS
    hs = head // NC
    chunkid = w // HEADS

    pltpu.sync_copy(pack1.at[head], pack_b)
    pltpu.sync_copy(zn, den_b)
    pltpu.sync_copy(zrows, numsh.at[pl.ds(s * 2500, 2500)])
    plsc.subcore_barrier()

    iota = lax.iota(jnp.int32, L)
    base_e = chunkid * EPT1

    def prep(k, ex_b, ig_b, id_b):
        eb = base_e + k * CH1
        pltpu.sync_copy(esrc.at[pl.ds(eb, CH1)], src_b)
        pltpu.sync_copy(edst.at[pl.ds(eb, CH1)], dst_b)

        @plsc.parallel_loop(0, CH1 // L, 1, unroll=2)
        def _(i):
            o = i * L
            sv = src_b[pl.ds(o, L)]
            dv = dst_b[pl.ds(o, L)]
            a_s = plsc.load_gather(pack_b, [sv])
            a_d = plsc.load_gather(pack_b, [dv + N])
            m_v = plsc.load_gather(pack_b, [dv + 2 * N])
            e = a_s + a_d
            e = jnp.where(e > 0, e, NEG_SLOPE * e)
            ex = jnp.exp(e - m_v)
            ex_b[pl.ds(o, L)] = ex
            ig_b[pl.ds(o, L)] = sv * HEADS + head
            id_b[pl.ds(o, L)] = dv + hs * N
            plsc.addupdate_scatter(den_b, [dv], ex)

    def scale_scatter(ex_b, rows_b, id_b):
        @plsc.parallel_loop(0, CH1 // L, 1, unroll=2)
        def _(i):
            pv = iota + i * L
            ex = ex_b[pl.ds(i * L, L)]
            for j in range(HID):
                jv = jnp.full((L,), j, jnp.int32)
                vals = plsc.load_gather(rows_b, [pv, jv])
                plsc.store_scatter(rows_b, [pv, jv], vals * ex)

        pltpu.sync_copy(rows_b, numsh.at[id_b], add=True)

    prep(0, ex0, ig0, id0)
    pltpu.async_copy(h1v.at[ig0], rows0, sem0)

    def pair(t, _):
        prep(2 * t + 1, ex1, ig1, id1)
        pltpu.async_copy(h1v.at[ig1], rows1, sem1)
        pltpu.make_async_copy(h1v.at[ig0], rows0, sem0).wait()
        scale_scatter(ex0, rows0, id0)

        @pl.when(t < NPAIR1 - 1)
        def _():
            prep(2 * t + 2, ex0, ig0, id0)
            pltpu.async_copy(h1v.at[ig0], rows0, sem0)

        pltpu.make_async_copy(h1v.at[ig1], rows1, sem1).wait()
        scale_scatter(ex1, rows1, id1)
        return 0

    lax.fori_loop(0, NPAIR1, pair, 0)
    pltpu.sync_copy(den_b, den1p.at[w])
    plsc.subcore_barrier()

    @pl.when(w < HEADS)
    def _():
        pltpu.sync_copy(numsh.at[pl.ds(hs * N, N)], num1.at[head])


def _sc1(h1v, pack1, esrc, edst, zrows, zn):
    mesh = plsc.VectorSubcoreMesh(core_axis_name="c", subcore_axis_name="s",
                                  num_cores=NC, num_subcores=NS)
    return pl.kernel(
        _sc1_body,
        out_type=(
            jax.ShapeDtypeStruct((HEADS, N, HID), jnp.float32),
            jax.ShapeDtypeStruct((NC * NS, N), jnp.float32),
        ),
        mesh=mesh,
        scratch_types=[
            pltpu.VMEM((3 * N,), jnp.float32),
            pltpu.VMEM((N,), jnp.float32),
            pltpu.VMEM((CH1,), jnp.int32),
            pltpu.VMEM((CH1,), jnp.int32),
            pltpu.VMEM((CH1,), jnp.float32),
            pltpu.VMEM((CH1,), jnp.float32),
            pltpu.VMEM((CH1,), jnp.int32),
            pltpu.VMEM((CH1,), jnp.int32),
            pltpu.VMEM((CH1,), jnp.int32),
            pltpu.VMEM((CH1,), jnp.int32),
            pltpu.VMEM((CH1, HID), jnp.float32),
            pltpu.VMEM((CH1, HID), jnp.float32),
            pltpu.VMEM_SHARED((4 * N, HID), jnp.float32),
            pltpu.SemaphoreType.DMA,
            pltpu.SemaphoreType.DMA,
        ],
        compiler_params=pltpu.CompilerParams(needs_layout_passes=False, use_tc_tiling_on_sc=False),
    )(h1v, pack1, esrc, edst, zrows, zn)


CH2 = 400
NCHUNK2 = 25
EPT2 = E // (NC * NS)


def _sc2_body(h2v, pack2, esrc, edst, zrows, zn, num2, den2p,
              pack_b, den_b, src_b, dst_b,
              ex0, ex1, ig0, ig1, id0, id1, rows0, rows1,
              numsh, sem0, sem1):
    c = lax.axis_index("c")
    s = lax.axis_index("s")
    w = s * NC + c

    pltpu.sync_copy(pack2, pack_b)
    pltpu.sync_copy(zn, den_b)
    pltpu.sync_copy(zrows.at[pl.ds(0, 1875)], numsh.at[pl.ds(s * 1875, 1875)])
    plsc.subcore_barrier()

    iota = lax.iota(jnp.int32, L)
    base_e = w * EPT2

    def prep(k, ex_b, ig_b, id_b):
        eb = base_e + k * CH2
        pltpu.sync_copy(esrc.at[pl.ds(eb, CH2)], src_b)
        pltpu.sync_copy(edst.at[pl.ds(eb, CH2)], dst_b)

        @plsc.parallel_loop(0, CH2 // L, 1, unroll=2)
        def _(i):
            o = i * L
            sv = src_b[pl.ds(o, L)]
            dv = dst_b[pl.ds(o, L)]
            a_s = plsc.load_gather(pack_b, [sv])
            a_d = plsc.load_gather(pack_b, [dv + N])
            m_v = plsc.load_gather(pack_b, [dv + 2 * N])
            e = a_s + a_d
            e = jnp.where(e > 0, e, NEG_SLOPE * e)
            ex = jnp.exp(e - m_v)
            ex_b[pl.ds(o, L)] = ex
            for g in range(NG):
                ig_b[pl.ds(g * CH2 + o, L)] = sv * NG + g
                id_b[pl.ds(g * CH2 + o, L)] = dv * NG + g
            plsc.addupdate_scatter(den_b, [dv], ex)

    def scale_scatter(ex_b, rows_b, id_b):
        @plsc.parallel_loop(0, CH2 // L, 1, unroll=2)
        def _(i):
            ex = ex_b[pl.ds(i * L, L)]
            for g in range(NG):
                pv = iota + (g * CH2 + i * L)
                for j in range(HID):
                    jv = jnp.full((L,), j, jnp.int32)
                    vals = plsc.load_gather(rows_b, [pv, jv])
                    plsc.store_scatter(rows_b, [pv, jv], vals * ex)

        pltpu.sync_copy(rows_b, numsh.at[id_b], add=True)

    prep(0, ex0, ig0, id0)
    pltpu.async_copy(h2v.at[ig0], rows0, sem0)

    def pair(t, _):
        prep(2 * t + 1, ex1, ig1, id1)
        pltpu.async_copy(h2v.at[ig1], rows1, sem1)
        pltpu.make_async_copy(h2v.at[ig0], rows0, sem0).wait()
        scale_scatter(ex0, rows0, id0)
        prep(2 * t + 2, ex0, ig0, id0)
        pltpu.async_copy(h2v.at[ig0], rows0, sem0)
        pltpu.make_async_copy(h2v.at[ig1], rows1, sem1).wait()
        scale_scatter(ex1, rows1, id1)
        return 0

    lax.fori_loop(0, NCHUNK2 // 2, pair, 0)
    pltpu.make_async_copy(h2v.at[ig0], rows0, sem0).wait()
    scale_scatter(ex0, rows0, id0)

    pltpu.sync_copy(den_b, den2p.at[w])
    plsc.subcore_barrier()

    @pl.when(s == 0)
    def _():
        pltpu.sync_copy(numsh, num2.at[c])


def _sc2(h2v, pack2, esrc, edst, zrows, zn):
    mesh = plsc.VectorSubcoreMesh(core_axis_name="c", subcore_axis_name="s",
                                  num_cores=NC, num_subcores=NS)
    return pl.kernel(
        _sc2_body,
        out_type=(
            jax.ShapeDtypeStruct((NC, NG * N, HID), jnp.float32),
            jax.ShapeDtypeStruct((NC * NS, N), jnp.float32),
        ),
        mesh=mesh,
        scratch_types=[
            pltpu.VMEM((3 * N,), jnp.float32),
            pltpu.VMEM((N,), jnp.float32),
            pltpu.VMEM((CH2,), jnp.int32),
            pltpu.VMEM((CH2,), jnp.int32),
            pltpu.VMEM((CH2,), jnp.float32),
            pltpu.VMEM((CH2,), jnp.float32),
            pltpu.VMEM((NG * CH2,), jnp.int32),
            pltpu.VMEM((NG * CH2,), jnp.int32),
            pltpu.VMEM((NG * CH2,), jnp.int32),
            pltpu.VMEM((NG * CH2,), jnp.int32),
            pltpu.VMEM((NG * CH2, HID), jnp.float32),
            pltpu.VMEM((NG * CH2, HID), jnp.float32),
            pltpu.VMEM_SHARED((NG * N, HID), jnp.float32),
            pltpu.SemaphoreType.DMA,
            pltpu.SemaphoreType.DMA,
        ],
        compiler_params=pltpu.CompilerParams(needs_layout_passes=False, use_tc_tiling_on_sc=False),
    )(h2v, pack2, esrc, edst, zrows, zn)


def kernel(x, edge_index, W1, a_src1, a_dst1, b1, W2, a_src2, a_dst2, b2):
    esrc = edge_index[0]
    edst = edge_index[1]
    eye = jnp.eye(HEADS, dtype=jnp.float32)
    a_s = (a_src1.reshape(HEADS, HID)[:, None, :]
           * eye[:, :, None]).transpose(0, 2, 1).reshape(HEADS * HID, HEADS)
    a_d = (a_dst1.reshape(HEADS, HID)[:, None, :]
           * eye[:, :, None]).transpose(0, 2, 1).reshape(HEADS * HID, HEADS)
    zrows = jnp.zeros((2500, HID), jnp.float32)
    zn = jnp.zeros((N,), jnp.float32)

    h1, asrcT, adstT = _tc_a1(x, W1, a_s, a_d)
    pack1 = _tc_a2(asrcT, adstT)
    h1v = h1.reshape(N * HEADS, HID)
    num1, den1p = _sc1(h1v, pack1.reshape(HEADS, 3 * N), esrc, edst, zrows, zn)

    w2p = jnp.pad(W2, ((0, 0), (0, NCP - NUM_CLASSES)))
    was2 = (W2 @ a_src2.reshape(NUM_CLASSES)).reshape(HEADS, HID)
    wad2 = (W2 @ a_dst2.reshape(NUM_CLASSES)).reshape(HEADS, HID)
    den1pt = den1p.reshape(NC * NS, N // BLK, BLK).transpose(1, 0, 2)
    h2, as2, ad2 = _tc_b1(num1, den1pt, b1.reshape(HEADS, HID), w2p, was2,
                          wad2)
    pack2 = _tc_b2(as2.reshape(1, N), ad2.reshape(1, N))

    h2v = h2.reshape(N * NG, HID)
    num2, den2p = _sc2(h2v, pack2.reshape(3 * N), esrc, edst, zrows, zn)

    return _tc_c(num2.reshape(NC, N, NCP), den2p, b2.reshape(1, NUM_CLASSES))
